# baseline (speedup 1.0000x reference)
import jax
import jax.numpy as jnp
from jax.experimental import pallas as pl
from jax.experimental.pallas import tpu as pltpu


def _seg_softmax(e, seg, n):
    m = jax.ops.segment_max(e, seg, num_segments=n)
    m = jnp.where(jnp.isfinite(m), m, 0.0)
    ex = jnp.exp(e - m[seg])
    den = jax.ops.segment_sum(ex, seg, num_segments=n)
    return ex / (den[seg] + 1e-16)


def _gat(x, src, dst, W, a_s, a_d, b, n):
    h = x @ W
    e = jax.nn.leaky_relu((h * a_s).sum(-1)[src] + (h * a_d).sum(-1)[dst],
                          negative_slope=0.2)
    a = _seg_softmax(e, dst, n)
    return jax.ops.segment_sum(a[:, None] * h[src], dst, num_segments=n) + b


def _gcn(x, ei, W1, b1, W2, b2, n):
    loop = jnp.arange(n, dtype=ei.dtype)
    ei2 = jnp.concatenate([ei, jnp.stack([loop, loop])], axis=1)
    src, dst = ei2[0], ei2[1]
    deg = jax.ops.segment_sum(jnp.ones(src.shape[0], jnp.float32), dst,
                              num_segments=n)
    dis = jnp.where(deg > 0, 1.0 / jnp.sqrt(deg), 0.0)
    norm = (dis[src] * dis[dst])[:, None]
    prop = lambda h: jax.ops.segment_sum(h[src] * norm, dst, num_segments=n)
    h = jax.nn.relu(prop(x @ W1) + b1)
    return prop(h @ W2) + b2


def _lap_evecs(ei, n, k=2):
    src, dst = ei[0], ei[1]
    w = jnp.ones(src.shape[0], jnp.float32)
    deg = jax.ops.segment_sum(w, src, num_segments=n)
    dis = jnp.where(deg > 0, 1.0 / jnp.sqrt(deg), 0.0)
    L = jnp.zeros((n, n), jnp.float32).at[src, dst].add(-w * dis[src] * dis[dst])
    L = L.at[jnp.arange(n), jnp.arange(n)].add(1.0)
    evals, evecs = jnp.linalg.eigh(L)
    return evecs[:, :k], evals[1]


def kernel(x, mask, edge_index, Wg1, bg1, Wg2, bg2, Wa1, as1, ad1, ba1,
           Wa2, as2, ad2, ba2, Wq_m, Wk_m, Wv_m, Wo_m, Wq_t, Wk_t, Wv_t,
           Ws_t, bq_m, bk_m, bv_m, bo_m, bq_t, bk_t, bv_t, bs_t, W1, b1,
           W2, b2, Wc, bc):
    n = x.shape[0]
    lap_ev, fiedler = _lap_evecs(edge_index, n, 2)
    x_combined = jnp.concatenate([x, lap_ev], axis=1)
    u = jax.lax.stop_gradient(_gcn(x_combined, edge_index, Wg1, bg1, Wg2, bg2, n))
    x_enriched = jnp.concatenate([x_combined, u], axis=1)
    loop = jnp.arange(n, dtype=edge_index.dtype)
    ei_sl = jnp.concatenate([edge_index, jnp.stack([loop, loop])], axis=1)
    src, dst = ei_sl[0], ei_sl[1]
    h = jax.nn.relu(_gat(x_combined, src, dst, Wa1, as1, ad1, ba1, n))
    h = _gat(h, src, dst, Wa2, as2, ad2, ba2, n)
    q = h @ Wq_m + bq_m
    k_ = h @ Wk_m + bk_m
    v = h @ Wv_m + bv_m
    att = jax.nn.softmax(q @ k_.T / jnp.sqrt(5.0), axis=-1)
    h = (att @ v) @ Wo_m + bo_m
    q = h @ Wq_t + bq_t
    k_ = h @ Wk_t + bk_t
    v = h @ Wv_t + bv_t
    e = (q[dst] * k_[src]).sum(-1) / jnp.sqrt(5.0)
    a = _seg_softmax(e, dst, n)
    h = jax.ops.segment_sum(a[:, None] * v[src], dst, num_segments=n) \
        + (h @ Ws_t + bs_t)
    h = pl.pallas_call(
        lambda i_ref, o_ref: o_ref.__setitem__(slice(None), i_ref[:]),
        out_shape=jax.ShapeDtypeStruct(h.shape, h.dtype))(h)
    P = jnp.zeros((n, n, 5), h.dtype).at[:, 0, :].set(h).reshape(n, n * 5)
    logits = jax.nn.relu(P @ W1 + b1) @ W2 + b2
    pu = _gcn(h, ei_sl, Wg1, bg1, Wg2, bg2, n)
    h_t = 0.8 - jnp.max(pu)
    crit = jnp.argmax(pu)
    logits = jnp.where(h_t < 0.2, logits.at[crit].add(10.0), logits)
    masked_logits = logits * mask
    value = jnp.mean(logits @ Wc + bc)
    return (masked_logits, value, x_combined, ei_sl)


# speedup vs baseline: 1.0154x; 1.0154x over previous
import jax
import jax.numpy as jnp
from jax.experimental import pallas as pl


_BLK = 256


def _dot(a, b):
    return jax.lax.dot_general(a, b, (((1,), (0,)), ((), ())),
                               precision=jax.lax.Precision.HIGHEST,
                               preferred_element_type=jnp.float32)


def _attn_body(h_blk, h_full, Wq, Wk, Wv, Wo, Wqt, Wkt, Wvt, Wst,
               bq, bk, bv, bo, bqt, bkt, bvt, bst,
               qt_o, kt_o, vt_o, st_o):
    hb = h_blk[:]
    hf = h_full[:]
    q = _dot(hb, Wq[:]) + bq[:]
    k = _dot(hf, Wk[:]) + bk[:]
    v = _dot(hf, Wv[:]) + bv[:]
    scores = jax.lax.dot_general(q, k, (((1,), (1,)), ((), ())),
                                 precision=jax.lax.Precision.HIGHEST,
                                 preferred_element_type=jnp.float32)
    scores = scores / jnp.sqrt(5.0)
    m = jnp.max(scores, axis=1, keepdims=True)
    e = jnp.exp(scores - m)
    att = e / jnp.sum(e, axis=1, keepdims=True)
    h2 = _dot(_dot(att, v), Wo[:]) + bo[:]
    qt_o[:] = _dot(h2, Wqt[:]) + bqt[:]
    kt_o[:] = _dot(h2, Wkt[:]) + bkt[:]
    vt_o[:] = _dot(h2, Wvt[:]) + bvt[:]
    st_o[:] = _dot(h2, Wst[:]) + bst[:]


def _attn_block(h, Wq, Wk, Wv, Wo, Wqt, Wkt, Wvt, Wst,
                bq, bk, bv, bo, bqt, bkt, bvt, bst):
    n, d = h.shape
    row = pl.BlockSpec((_BLK, d), lambda i: (i, 0))
    full = pl.BlockSpec(h.shape, lambda i: (0, 0))
    wspec = pl.BlockSpec((d, d), lambda i: (0, 0))
    bspec = pl.BlockSpec((1, d), lambda i: (0, 0))
    out = jax.ShapeDtypeStruct((n, d), jnp.float32)
    b2d = lambda b: b.reshape(1, d)
    return pl.pallas_call(
        _attn_body,
        grid=(n // _BLK,),
        in_specs=[row, full] + [wspec] * 8 + [bspec] * 8,
        out_specs=[row] * 4,
        out_shape=[out] * 4,
    )(h, h, Wq, Wk, Wv, Wo, Wqt, Wkt, Wvt, Wst,
      b2d(bq), b2d(bk), b2d(bv), b2d(bo),
      b2d(bqt), b2d(bkt), b2d(bvt), b2d(bst))


def _mlp_body(h_blk, W1, b1, W2, b2, out):
    a = jnp.maximum(_dot(h_blk[:], W1[:]) + b1[:], 0.0)
    out[:] = _dot(a, W2[:]) + b2[:]


def _mlp_block(h, W1_5, b1, W2, b2):
    n, d = h.shape
    hdim = W1_5.shape[1]
    return pl.pallas_call(
        _mlp_body,
        grid=(n // _BLK,),
        in_specs=[pl.BlockSpec((_BLK, d), lambda i: (i, 0)),
                  pl.BlockSpec((d, hdim), lambda i: (0, 0)),
                  pl.BlockSpec((1, hdim), lambda i: (0, 0)),
                  pl.BlockSpec((hdim, n), lambda i: (0, 0)),
                  pl.BlockSpec((1, n), lambda i: (0, 0))],
        out_specs=pl.BlockSpec((_BLK, n), lambda i: (i, 0)),
        out_shape=jax.ShapeDtypeStruct((n, n), jnp.float32),
    )(h, W1_5, b1.reshape(1, hdim), W2, b2.reshape(1, n))


def _seg_softmax(e, seg, n):
    m = jax.ops.segment_max(e, seg, num_segments=n)
    m = jnp.where(jnp.isfinite(m), m, 0.0)
    ex = jnp.exp(e - m[seg])
    den = jax.ops.segment_sum(ex, seg, num_segments=n)
    return ex / (den[seg] + 1e-16)


def _gat(x, src, dst, W, a_s, a_d, b, n):
    h = x @ W
    e = jax.nn.leaky_relu((h * a_s).sum(-1)[src] + (h * a_d).sum(-1)[dst],
                          negative_slope=0.2)
    a = _seg_softmax(e, dst, n)
    return jax.ops.segment_sum(a[:, None] * h[src], dst, num_segments=n) + b


def _gcn(x, ei, W1, b1, W2, b2, n):
    loop = jnp.arange(n, dtype=ei.dtype)
    ei2 = jnp.concatenate([ei, jnp.stack([loop, loop])], axis=1)
    src, dst = ei2[0], ei2[1]
    deg = jax.ops.segment_sum(jnp.ones(src.shape[0], jnp.float32), dst,
                              num_segments=n)
    dis = jnp.where(deg > 0, 1.0 / jnp.sqrt(deg), 0.0)
    norm = (dis[src] * dis[dst])[:, None]
    prop = lambda h: jax.ops.segment_sum(h[src] * norm, dst, num_segments=n)
    h = jax.nn.relu(prop(x @ W1) + b1)
    return prop(h @ W2) + b2


def _lap_evecs(ei, n, k=2):
    src, dst = ei[0], ei[1]
    w = jnp.ones(src.shape[0], jnp.float32)
    deg = jax.ops.segment_sum(w, src, num_segments=n)
    dis = jnp.where(deg > 0, 1.0 / jnp.sqrt(deg), 0.0)
    L = jnp.zeros((n, n), jnp.float32).at[src, dst].add(-w * dis[src] * dis[dst])
    L = L.at[jnp.arange(n), jnp.arange(n)].add(1.0)
    evals, evecs = jnp.linalg.eigh(L)
    return evecs[:, :k], evals[1]


def kernel(x, mask, edge_index, Wg1, bg1, Wg2, bg2, Wa1, as1, ad1, ba1,
           Wa2, as2, ad2, ba2, Wq_m, Wk_m, Wv_m, Wo_m, Wq_t, Wk_t, Wv_t,
           Ws_t, bq_m, bk_m, bv_m, bo_m, bq_t, bk_t, bv_t, bs_t, W1, b1,
           W2, b2, Wc, bc):
    n = x.shape[0]
    lap_ev, _ = _lap_evecs(edge_index, n, 2)
    x_combined = jnp.concatenate([x, lap_ev], axis=1)
    loop = jnp.arange(n, dtype=edge_index.dtype)
    ei_sl = jnp.concatenate([edge_index, jnp.stack([loop, loop])], axis=1)
    src, dst = ei_sl[0], ei_sl[1]
    h = jax.nn.relu(_gat(x_combined, src, dst, Wa1, as1, ad1, ba1, n))
    h = _gat(h, src, dst, Wa2, as2, ad2, ba2, n)
    q, k_, v, skip = _attn_block(h, Wq_m, Wk_m, Wv_m, Wo_m,
                                 Wq_t, Wk_t, Wv_t, Ws_t,
                                 bq_m, bk_m, bv_m, bo_m,
                                 bq_t, bk_t, bv_t, bs_t)
    e = (q[dst] * k_[src]).sum(-1) / jnp.sqrt(5.0)
    a = _seg_softmax(e, dst, n)
    h = jax.ops.segment_sum(a[:, None] * v[src], dst, num_segments=n) + skip
    logits = _mlp_block(h, W1[:5], b1, W2, b2)
    pu = _gcn(h, ei_sl, Wg1, bg1, Wg2, bg2, n)
    h_t = 0.8 - jnp.max(pu)
    crit = jnp.argmax(pu)
    logits = jnp.where(h_t < 0.2, logits.at[crit].add(10.0), logits)
    masked_logits = logits * mask
    value = jnp.mean(logits @ Wc + bc)
    return (masked_logits, value, x_combined, ei_sl)
